# baseline (speedup 1.0000x reference)
import jax
import jax.numpy as jnp
from jax.experimental import pallas as pl
from jax.experimental.pallas import tpu as pltpu


def _proj1_kernel(f_ref, w_ref, b_ref, h_ref, res_ref, g_ref):
    cout = h_ref.shape[2]
    p = jnp.dot(f_ref[0], w_ref[...], preferred_element_type=jnp.float32)
    h_ref[0] = p[:, :cout] + b_ref[...]
    res_ref[0] = p[:, cout:2 * cout]
    g_ref[0] = p[:, 2 * cout:]


def _proj2_kernel(f_ref, w_ref, b_ref, h_ref, g_ref):
    cout = h_ref.shape[2]
    p = jnp.dot(f_ref[0], w_ref[...], preferred_element_type=jnp.float32)
    h_ref[0] = p[:, :cout] + b_ref[...]
    g_ref[0] = p[:, cout:]


def _gmax1_kernel(g_ref, idx_ref, h_ref, out_ref):
    kn = idx_ref.shape[2]

    def body(qb, carry):
        q0 = qb * 8
        accs = []
        for i in range(8):
            acc = g_ref[idx_ref[0, q0 + i, 0]]
            for k in range(1, kn):
                acc = jnp.maximum(acc, g_ref[idx_ref[0, q0 + i, k]])
            accs.append(acc)
        for i in range(8):
            out_ref[q0 + i] = jnp.maximum(h_ref[q0 + i] + accs[i], 0.0)
        return carry

    jax.lax.fori_loop(0, idx_ref.shape[1] // 8, body, 0)


def _gmax2_kernel(g_ref, idx_ref, h_ref, res_ref, out_ref):
    kn = idx_ref.shape[2]

    def body(qb, carry):
        q0 = qb * 8
        accs = []
        for i in range(8):
            acc = g_ref[idx_ref[0, q0 + i, 0]]
            for k in range(1, kn):
                acc = jnp.maximum(acc, g_ref[idx_ref[0, q0 + i, k]])
            accs.append(acc)
        for i in range(8):
            out_ref[q0 + i] = jnp.maximum(
                h_ref[q0 + i] + accs[i] + res_ref[q0 + i], 0.0)
        return carry

    jax.lax.fori_loop(0, idx_ref.shape[1] // 8, body, 0)


def _upsample_kernel(f_ref, r_ref, idx_ref, w_ref, fo_ref, ro_ref):
    def body(qb, carry):
        q0 = qb * 8
        fos, ros = [], []
        for i in range(8):
            i0 = idx_ref[0, q0 + i, 0]
            i1 = idx_ref[0, q0 + i, 1]
            i2 = idx_ref[0, q0 + i, 2]
            w0 = w_ref[0, q0 + i, 0]
            w1 = w_ref[0, q0 + i, 1]
            w2 = w_ref[0, q0 + i, 2]
            fos.append(f_ref[i0] * w0 + f_ref[i1] * w1 + f_ref[i2] * w2)
            ros.append(r_ref[i0] * w0 + r_ref[i1] * w1 + r_ref[i2] * w2)
        for i in range(8):
            fo_ref[q0 + i] = fos[i]
            ro_ref[q0 + i] = ros[i]
        return carry

    jax.lax.fori_loop(0, idx_ref.shape[1] // 8, body, 0)


_TB = 128
_CH = 128


def _ballq_kernel(ptsT_ref, q_ref, out_ref, idx_acc, carry_ref, smin_ref):
    kn = out_ref.shape[2]
    tb = q_ref.shape[1]
    p = ptsT_ref.shape[2]

    idx_acc[...] = jnp.full((tb, kn), -1.0, jnp.float32)
    carry_ref[...] = jnp.zeros((tb, _CH), jnp.float32)
    smin_ref[0] = 0.0

    qx = q_ref[0, :, 0:1]
    qy = q_ref[0, :, 1:2]
    qz = q_ref[0, :, 2:3]

    lt = (jax.lax.broadcasted_iota(jnp.int32, (_CH, _CH), 0)
          <= jax.lax.broadcasted_iota(jnp.int32, (_CH, _CH), 1)
          ).astype(jnp.float32)
    lane_iota = jax.lax.broadcasted_iota(
        jnp.int32, (tb, _CH), 1).astype(jnp.float32)

    def body(chunk, carry_in):
        @pl.when(smin_ref[0] < kn)
        def _():
            base = pl.multiple_of(chunk * _CH, _CH)
            x0 = ptsT_ref[0, 0:1, pl.ds(base, _CH)]
            y0 = ptsT_ref[0, 1:2, pl.ds(base, _CH)]
            z0 = ptsT_ref[0, 2:3, pl.ds(base, _CH)]
            dx = qx - x0
            dy = qy - y0
            dz = qz - z0
            d2 = (dx * dx + dy * dy) + dz * dz
            within = (d2 < 0.5625).astype(jnp.float32)
            r = jnp.dot(within, lt, preferred_element_type=jnp.float32)
            carry = carry_ref[...]
            grank = jnp.where(within > 0.5, carry + r, 0.0)
            jglob = lane_iota + chunk.astype(jnp.float32) * _CH
            for g in range(kn // 8):
                @pl.when(smin_ref[0] < (g + 1) * 8.0)
                def _(g=g):
                    vals = [jnp.max(jnp.where(grank == (s + 1.0), jglob, -1.0),
                                    axis=-1, keepdims=True)
                            for s in range(g * 8, (g + 1) * 8)]
                    contrib = jnp.concatenate(vals, axis=1)
                    cur = idx_acc[:, g * 8:(g + 1) * 8]
                    idx_acc[:, g * 8:(g + 1) * 8] = jnp.maximum(cur, contrib)
            new_carry = carry + jnp.sum(within, axis=-1, keepdims=True)
            carry_ref[...] = new_carry
            smin_ref[0] = jnp.min(new_carry)
        return carry_in

    jax.lax.fori_loop(0, p // _CH, body, 0)

    cnt = carry_ref[...][:, :kn]
    acc = idx_acc[...]
    slot = jax.lax.broadcasted_iota(jnp.int32, (tb, kn), 1).astype(jnp.float32)
    final = jnp.where(slot < cnt, acc, acc[:, 0:1])
    out_ref[0] = final.astype(jnp.int32)


def _three_nn_kernel(ptsT_ref, q_ref, idx_ref, w_ref, d2s_ref):
    tb = q_ref.shape[1]
    p = ptsT_ref.shape[2]
    nch = p // _CH

    qx = q_ref[0, :, 0:1]
    qy = q_ref[0, :, 1:2]
    qz = q_ref[0, :, 2:3]
    for chunk in range(nch):
        x0 = ptsT_ref[0, 0:1, chunk * _CH:(chunk + 1) * _CH]
        y0 = ptsT_ref[0, 1:2, chunk * _CH:(chunk + 1) * _CH]
        z0 = ptsT_ref[0, 2:3, chunk * _CH:(chunk + 1) * _CH]
        dx = qx - x0
        dy = qy - y0
        dz = qz - z0
        d2s_ref[:, chunk * _CH:(chunk + 1) * _CH] = (dx * dx + dy * dy) + dz * dz

    big = jnp.float32(3.0e38)
    found_i = []
    found_v = []
    for r in range(3):
        acc = None
        for chunk in range(nch):
            d2c = d2s_ref[:, chunk * _CH:(chunk + 1) * _CH]
            jglob = (jax.lax.broadcasted_iota(jnp.int32, (tb, _CH), 1)
                     .astype(jnp.float32) + jnp.float32(chunk * _CH))
            for fi in found_i:
                d2c = jnp.where(jglob == fi, big, d2c)
            acc = d2c if acc is None else jnp.minimum(acc, d2c)
        gmin = jnp.min(acc, axis=-1, keepdims=True)
        iacc = None
        for chunk in range(nch):
            d2c = d2s_ref[:, chunk * _CH:(chunk + 1) * _CH]
            jglob = (jax.lax.broadcasted_iota(jnp.int32, (tb, _CH), 1)
                     .astype(jnp.float32) + jnp.float32(chunk * _CH))
            for fi in found_i:
                d2c = jnp.where(jglob == fi, big, d2c)
            cand = jnp.where(d2c == gmin, jglob, big)
            iacc = cand if iacc is None else jnp.minimum(iacc, cand)
        gidx = jnp.min(iacc, axis=-1, keepdims=True)
        found_i.append(gidx)
        found_v.append(gmin)

    w = [1.0 / (v + 1e-8) for v in found_v]
    wsum = (w[0] + w[1]) + w[2]
    w = [x / wsum for x in w]
    idx_ref[0] = jnp.concatenate(found_i, axis=1).astype(jnp.int32)
    w_ref[0] = jnp.concatenate(w, axis=1)


def _ball_query_idx(ptsT, pts4, k):
    bs, _, p = ptsT.shape
    nt = p // _TB
    return pl.pallas_call(
        _ballq_kernel,
        grid=(bs, nt),
        in_specs=[pl.BlockSpec((1, 8, p), lambda b, t: (b, 0, 0)),
                  pl.BlockSpec((1, _TB, 4), lambda b, t: (b, t, 0))],
        out_specs=pl.BlockSpec((1, _TB, k), lambda b, t: (b, t, 0)),
        out_shape=jax.ShapeDtypeStruct((bs, p, k), jnp.int32),
        scratch_shapes=[pltpu.VMEM((_TB, k), jnp.float32),
                        pltpu.VMEM((_TB, _CH), jnp.float32),
                        pltpu.SMEM((1,), jnp.float32)],
        compiler_params=pltpu.CompilerParams(
            dimension_semantics=("parallel", "parallel")),
    )(ptsT, pts4)


def _three_nn(ptsT_child, pts4_parent):
    bs, _, pc = ptsT_child.shape
    pp = pts4_parent.shape[1]
    nt = pp // _TB
    return pl.pallas_call(
        _three_nn_kernel,
        grid=(bs, nt),
        in_specs=[pl.BlockSpec((1, 8, pc), lambda b, t: (b, 0, 0)),
                  pl.BlockSpec((1, _TB, 4), lambda b, t: (b, t, 0))],
        out_specs=[pl.BlockSpec((1, _TB, 3), lambda b, t: (b, t, 0)),
                   pl.BlockSpec((1, _TB, 3), lambda b, t: (b, t, 0))],
        out_shape=[jax.ShapeDtypeStruct((bs, pp, 3), jnp.int32),
                   jax.ShapeDtypeStruct((bs, pp, 3), jnp.float32)],
        scratch_shapes=[pltpu.VMEM((_TB, pc), jnp.float32)],
        compiler_params=pltpu.CompilerParams(
            dimension_semantics=("parallel", "parallel")),
    )(ptsT_child, pts4_parent)


def _coord_views(pts):
    ptsT = jnp.pad(pts.transpose(0, 2, 1), ((0, 0), (0, 5), (0, 0)))
    pts4 = jnp.pad(pts, ((0, 0), (0, 0), (0, 1)))
    return ptsT, pts4


_TQ = 256


def _row_spec3(nrows, cols, index_map):
    return pl.BlockSpec((nrows, 1, cols), index_map)


def kernel(xyz, parent_xyz, feats, w1, b1, w2, b2, w_res):
    bs, c, p0 = feats.shape
    p1 = parent_xyz.shape[1]
    cout = w1.shape[0]
    kn = 32
    nt0 = p0 // _TQ
    nt1 = p1 // _TQ

    cparams = pltpu.CompilerParams(
        dimension_semantics=("parallel", "parallel"))
    cparams1 = pltpu.CompilerParams(dimension_semantics=("parallel",))

    xyzT, xyz4 = _coord_views(xyz)
    pxyzT, pxyz4 = _coord_views(parent_xyz)
    idx1 = _ball_query_idx(xyzT, xyz4, kn)
    idx2 = _ball_query_idx(pxyzT, pxyz4, kn)
    idx3, w3 = _three_nn(xyzT, pxyz4)

    feats_t = feats.transpose(0, 2, 1)
    w1a, w1b = w1[:, :c], w1[:, c:]
    w1cat = jnp.concatenate([w1a - w1b, w_res, w1b], 0)
    h1, res1, g1 = pl.pallas_call(
        _proj1_kernel,
        grid=(bs,),
        in_specs=[pl.BlockSpec((1, p0, c), lambda b: (b, 0, 0)),
                  pl.BlockSpec((c, 3 * cout), lambda b: (0, 0)),
                  pl.BlockSpec((1, cout), lambda b: (0, 0))],
        out_specs=[pl.BlockSpec((1, p0, cout), lambda b: (b, 0, 0))] * 3,
        out_shape=[jax.ShapeDtypeStruct((bs, p0, cout), jnp.float32)] * 3,
        compiler_params=cparams1,
    )(feats_t, w1cat.T, b1[None, :])

    feats1 = pl.pallas_call(
        _gmax1_kernel,
        grid=(bs, nt0),
        in_specs=[_row_spec3(p0, cout, lambda b, t: (b, 0, 0)),
                  pl.BlockSpec((1, _TQ, kn), lambda b, t: (b * nt0 + t, 0, 0),
                               memory_space=pltpu.SMEM),
                  _row_spec3(_TQ, cout, lambda b, t: (b * nt0 + t, 0, 0))],
        out_specs=_row_spec3(_TQ, cout, lambda b, t: (b * nt0 + t, 0, 0)),
        out_shape=jax.ShapeDtypeStruct((bs * p0, 1, cout), jnp.float32),
        compiler_params=cparams,
    )(g1.reshape(bs * p0, 1, cout),
      idx1.reshape(bs * nt0, _TQ, kn),
      h1.reshape(bs * p0, 1, cout))

    f_up, r_up = pl.pallas_call(
        _upsample_kernel,
        grid=(bs, nt1),
        in_specs=[_row_spec3(p0, cout, lambda b, t: (b, 0, 0)),
                  _row_spec3(p0, cout, lambda b, t: (b, 0, 0)),
                  pl.BlockSpec((1, _TQ, 3), lambda b, t: (b * nt1 + t, 0, 0),
                               memory_space=pltpu.SMEM),
                  pl.BlockSpec((1, _TQ, 3), lambda b, t: (b * nt1 + t, 0, 0),
                               memory_space=pltpu.SMEM)],
        out_specs=[_row_spec3(_TQ, cout, lambda b, t: (b * nt1 + t, 0, 0))] * 2,
        out_shape=[jax.ShapeDtypeStruct((bs * p1, 1, cout), jnp.float32)] * 2,
        compiler_params=cparams,
    )(feats1,
      res1.reshape(bs * p0, 1, cout),
      idx3.reshape(bs * nt1, _TQ, 3),
      w3.reshape(bs * nt1, _TQ, 3))

    w2a, w2b = w2[:, :cout], w2[:, cout:]
    w2cat = jnp.concatenate([w2a - w2b, w2b], 0)
    h2, g2 = pl.pallas_call(
        _proj2_kernel,
        grid=(bs,),
        in_specs=[pl.BlockSpec((1, p1, cout), lambda b: (b, 0, 0)),
                  pl.BlockSpec((cout, 2 * cout), lambda b: (0, 0)),
                  pl.BlockSpec((1, cout), lambda b: (0, 0))],
        out_specs=[pl.BlockSpec((1, p1, cout), lambda b: (b, 0, 0))] * 2,
        out_shape=[jax.ShapeDtypeStruct((bs, p1, cout), jnp.float32)] * 2,
        compiler_params=cparams1,
    )(f_up.reshape(bs, p1, cout), w2cat.T, b2[None, :])

    out_rows = pl.pallas_call(
        _gmax2_kernel,
        grid=(bs, nt1),
        in_specs=[_row_spec3(p1, cout, lambda b, t: (b, 0, 0)),
                  pl.BlockSpec((1, _TQ, kn), lambda b, t: (b * nt1 + t, 0, 0),
                               memory_space=pltpu.SMEM),
                  _row_spec3(_TQ, cout, lambda b, t: (b * nt1 + t, 0, 0)),
                  _row_spec3(_TQ, cout, lambda b, t: (b * nt1 + t, 0, 0))],
        out_specs=_row_spec3(_TQ, cout, lambda b, t: (b * nt1 + t, 0, 0)),
        out_shape=jax.ShapeDtypeStruct((bs * p1, 1, cout), jnp.float32),
        compiler_params=cparams,
    )(g2.reshape(bs * p1, 1, cout),
      idx2.reshape(bs * nt1, _TQ, kn),
      h2.reshape(bs * p1, 1, cout),
      r_up)

    out = out_rows.reshape(bs, p1, cout).transpose(0, 2, 1)
    return parent_xyz, out


# speedup vs baseline: 1.1267x; 1.1267x over previous
import jax
import jax.numpy as jnp
from jax.experimental import pallas as pl
from jax.experimental.pallas import tpu as pltpu


def _proj1_kernel(f_ref, w_ref, b_ref, h_ref, res_ref, g_ref):
    cout = h_ref.shape[2]
    p = jnp.dot(f_ref[0], w_ref[...], preferred_element_type=jnp.float32)
    h_ref[0] = p[:, :cout] + b_ref[...]
    res_ref[0] = p[:, cout:2 * cout]
    g_ref[0] = p[:, 2 * cout:]


def _proj2_kernel(f_ref, w_ref, b_ref, h_ref, g_ref):
    cout = h_ref.shape[2]
    p = jnp.dot(f_ref[0], w_ref[...], preferred_element_type=jnp.float32)
    h_ref[0] = p[:, :cout] + b_ref[...]
    g_ref[0] = p[:, cout:]


def _gmax1_kernel(g_ref, idx_ref, h_ref, out_ref):
    kn = idx_ref.shape[2]

    def body(qb, carry):
        q0 = qb * 8
        accs = []
        for i in range(8):
            acc = g_ref[idx_ref[0, q0 + i, 0]]
            for k in range(1, kn):
                acc = jnp.maximum(acc, g_ref[idx_ref[0, q0 + i, k]])
            accs.append(acc)
        for i in range(8):
            out_ref[q0 + i] = jnp.maximum(h_ref[q0 + i] + accs[i], 0.0)
        return carry

    jax.lax.fori_loop(0, idx_ref.shape[1] // 8, body, 0)


def _gmax2_kernel(g_ref, idx_ref, h_ref, res_ref, out_ref):
    kn = idx_ref.shape[2]

    def body(qb, carry):
        q0 = qb * 8
        accs = []
        for i in range(8):
            acc = g_ref[idx_ref[0, q0 + i, 0]]
            for k in range(1, kn):
                acc = jnp.maximum(acc, g_ref[idx_ref[0, q0 + i, k]])
            accs.append(acc)
        for i in range(8):
            out_ref[q0 + i] = jnp.maximum(
                h_ref[q0 + i] + accs[i] + res_ref[q0 + i], 0.0)
        return carry

    jax.lax.fori_loop(0, idx_ref.shape[1] // 8, body, 0)


def _upsample_kernel(f_ref, r_ref, idx_ref, w_ref, fo_ref, ro_ref):
    def body(qb, carry):
        q0 = qb * 8
        fos, ros = [], []
        for i in range(8):
            i0 = idx_ref[0, q0 + i, 0]
            i1 = idx_ref[0, q0 + i, 1]
            i2 = idx_ref[0, q0 + i, 2]
            w0 = w_ref[0, q0 + i, 0]
            w1 = w_ref[0, q0 + i, 1]
            w2 = w_ref[0, q0 + i, 2]
            fos.append(f_ref[i0] * w0 + f_ref[i1] * w1 + f_ref[i2] * w2)
            ros.append(r_ref[i0] * w0 + r_ref[i1] * w1 + r_ref[i2] * w2)
        for i in range(8):
            fo_ref[q0 + i] = fos[i]
            ro_ref[q0 + i] = ros[i]
        return carry

    jax.lax.fori_loop(0, idx_ref.shape[1] // 8, body, 0)


_TB = 256
_CH = 128


def _ballq_kernel(ptsT_ref, q_ref, out_ref, idx_acc, carry_ref, smin_ref):
    kn = out_ref.shape[2]
    tb = q_ref.shape[1]
    p = ptsT_ref.shape[2]

    idx_acc[...] = jnp.full((tb, kn), -1.0, jnp.float32)
    carry_ref[...] = jnp.zeros((tb, _CH), jnp.float32)
    smin_ref[0] = 0.0

    qx = q_ref[0, :, 0:1]
    qy = q_ref[0, :, 1:2]
    qz = q_ref[0, :, 2:3]

    lt = (jax.lax.broadcasted_iota(jnp.int32, (_CH, _CH), 0)
          <= jax.lax.broadcasted_iota(jnp.int32, (_CH, _CH), 1)
          ).astype(jnp.float32)
    lane_iota = jax.lax.broadcasted_iota(
        jnp.int32, (tb, _CH), 1).astype(jnp.float32)

    def body(chunk, carry_in):
        @pl.when(smin_ref[0] < kn)
        def _():
            base = pl.multiple_of(chunk * _CH, _CH)
            x0 = ptsT_ref[0, 0:1, pl.ds(base, _CH)]
            y0 = ptsT_ref[0, 1:2, pl.ds(base, _CH)]
            z0 = ptsT_ref[0, 2:3, pl.ds(base, _CH)]
            dx = qx - x0
            dy = qy - y0
            dz = qz - z0
            d2 = (dx * dx + dy * dy) + dz * dz
            within = (d2 < 0.5625).astype(jnp.float32)
            r = jnp.dot(within, lt, preferred_element_type=jnp.float32)
            carry = carry_ref[...]
            grank = jnp.where(within > 0.5, carry + r, 0.0)
            jglob = lane_iota + chunk.astype(jnp.float32) * _CH
            vals = [jnp.max(jnp.where(grank == (s + 1.0), jglob, -1.0),
                            axis=-1, keepdims=True) for s in range(kn)]
            contrib = jnp.concatenate(vals, axis=1)
            idx_acc[...] = jnp.maximum(idx_acc[...], contrib)
            new_carry = carry + jnp.sum(within, axis=-1, keepdims=True)
            carry_ref[...] = new_carry
            smin_ref[0] = jnp.min(new_carry)
        return carry_in

    jax.lax.fori_loop(0, p // _CH, body, 0)

    cnt = carry_ref[...][:, :kn]
    acc = idx_acc[...]
    slot = jax.lax.broadcasted_iota(jnp.int32, (tb, kn), 1).astype(jnp.float32)
    final = jnp.where(slot < cnt, acc, acc[:, 0:1])
    out_ref[0] = final.astype(jnp.int32)


def _three_nn_kernel(ptsT_ref, q_ref, idx_ref, w_ref, d2s_ref):
    tb = q_ref.shape[1]
    p = ptsT_ref.shape[2]
    nch = p // _CH

    qx = q_ref[0, :, 0:1]
    qy = q_ref[0, :, 1:2]
    qz = q_ref[0, :, 2:3]
    for chunk in range(nch):
        x0 = ptsT_ref[0, 0:1, chunk * _CH:(chunk + 1) * _CH]
        y0 = ptsT_ref[0, 1:2, chunk * _CH:(chunk + 1) * _CH]
        z0 = ptsT_ref[0, 2:3, chunk * _CH:(chunk + 1) * _CH]
        dx = qx - x0
        dy = qy - y0
        dz = qz - z0
        d2s_ref[:, chunk * _CH:(chunk + 1) * _CH] = (dx * dx + dy * dy) + dz * dz

    big = jnp.float32(3.0e38)
    found_i = []
    found_v = []
    for r in range(3):
        acc = None
        for chunk in range(nch):
            d2c = d2s_ref[:, chunk * _CH:(chunk + 1) * _CH]
            jglob = (jax.lax.broadcasted_iota(jnp.int32, (tb, _CH), 1)
                     .astype(jnp.float32) + jnp.float32(chunk * _CH))
            for fi in found_i:
                d2c = jnp.where(jglob == fi, big, d2c)
            acc = d2c if acc is None else jnp.minimum(acc, d2c)
        gmin = jnp.min(acc, axis=-1, keepdims=True)
        iacc = None
        for chunk in range(nch):
            d2c = d2s_ref[:, chunk * _CH:(chunk + 1) * _CH]
            jglob = (jax.lax.broadcasted_iota(jnp.int32, (tb, _CH), 1)
                     .astype(jnp.float32) + jnp.float32(chunk * _CH))
            for fi in found_i:
                d2c = jnp.where(jglob == fi, big, d2c)
            cand = jnp.where(d2c == gmin, jglob, big)
            iacc = cand if iacc is None else jnp.minimum(iacc, cand)
        gidx = jnp.min(iacc, axis=-1, keepdims=True)
        found_i.append(gidx)
        found_v.append(gmin)

    w = [1.0 / (v + 1e-8) for v in found_v]
    wsum = (w[0] + w[1]) + w[2]
    w = [x / wsum for x in w]
    idx_ref[0] = jnp.concatenate(found_i, axis=1).astype(jnp.int32)
    w_ref[0] = jnp.concatenate(w, axis=1)


def _ball_query_idx(ptsT, pts4, k):
    bs, _, p = ptsT.shape
    nt = p // _TB
    return pl.pallas_call(
        _ballq_kernel,
        grid=(bs, nt),
        in_specs=[pl.BlockSpec((1, 8, p), lambda b, t: (b, 0, 0)),
                  pl.BlockSpec((1, _TB, 4), lambda b, t: (b, t, 0))],
        out_specs=pl.BlockSpec((1, _TB, k), lambda b, t: (b, t, 0)),
        out_shape=jax.ShapeDtypeStruct((bs, p, k), jnp.int32),
        scratch_shapes=[pltpu.VMEM((_TB, k), jnp.float32),
                        pltpu.VMEM((_TB, _CH), jnp.float32),
                        pltpu.SMEM((1,), jnp.float32)],
        compiler_params=pltpu.CompilerParams(
            dimension_semantics=("parallel", "parallel")),
    )(ptsT, pts4)


def _three_nn(ptsT_child, pts4_parent):
    bs, _, pc = ptsT_child.shape
    pp = pts4_parent.shape[1]
    nt = pp // _TB
    return pl.pallas_call(
        _three_nn_kernel,
        grid=(bs, nt),
        in_specs=[pl.BlockSpec((1, 8, pc), lambda b, t: (b, 0, 0)),
                  pl.BlockSpec((1, _TB, 4), lambda b, t: (b, t, 0))],
        out_specs=[pl.BlockSpec((1, _TB, 3), lambda b, t: (b, t, 0)),
                   pl.BlockSpec((1, _TB, 3), lambda b, t: (b, t, 0))],
        out_shape=[jax.ShapeDtypeStruct((bs, pp, 3), jnp.int32),
                   jax.ShapeDtypeStruct((bs, pp, 3), jnp.float32)],
        scratch_shapes=[pltpu.VMEM((_TB, pc), jnp.float32)],
        compiler_params=pltpu.CompilerParams(
            dimension_semantics=("parallel", "parallel")),
    )(ptsT_child, pts4_parent)


def _coord_views(pts):
    ptsT = jnp.pad(pts.transpose(0, 2, 1), ((0, 0), (0, 5), (0, 0)))
    pts4 = jnp.pad(pts, ((0, 0), (0, 0), (0, 1)))
    return ptsT, pts4


_TQ = 256


def _row_spec3(nrows, cols, index_map):
    return pl.BlockSpec((nrows, 1, cols), index_map)


def kernel(xyz, parent_xyz, feats, w1, b1, w2, b2, w_res):
    bs, c, p0 = feats.shape
    p1 = parent_xyz.shape[1]
    cout = w1.shape[0]
    kn = 32
    nt0 = p0 // _TQ
    nt1 = p1 // _TQ

    cparams = pltpu.CompilerParams(
        dimension_semantics=("parallel", "parallel"))
    cparams1 = pltpu.CompilerParams(dimension_semantics=("parallel",))

    xyzT, xyz4 = _coord_views(xyz)
    pxyzT, pxyz4 = _coord_views(parent_xyz)
    idx1 = _ball_query_idx(xyzT, xyz4, kn)
    idx2 = _ball_query_idx(pxyzT, pxyz4, kn)
    idx3, w3 = _three_nn(xyzT, pxyz4)

    feats_t = feats.transpose(0, 2, 1)
    w1a, w1b = w1[:, :c], w1[:, c:]
    w1cat = jnp.concatenate([w1a - w1b, w_res, w1b], 0)
    h1, res1, g1 = pl.pallas_call(
        _proj1_kernel,
        grid=(bs,),
        in_specs=[pl.BlockSpec((1, p0, c), lambda b: (b, 0, 0)),
                  pl.BlockSpec((c, 3 * cout), lambda b: (0, 0)),
                  pl.BlockSpec((1, cout), lambda b: (0, 0))],
        out_specs=[pl.BlockSpec((1, p0, cout), lambda b: (b, 0, 0))] * 3,
        out_shape=[jax.ShapeDtypeStruct((bs, p0, cout), jnp.float32)] * 3,
        compiler_params=cparams1,
    )(feats_t, w1cat.T, b1[None, :])

    feats1 = pl.pallas_call(
        _gmax1_kernel,
        grid=(bs, nt0),
        in_specs=[_row_spec3(p0, cout, lambda b, t: (b, 0, 0)),
                  pl.BlockSpec((1, _TQ, kn), lambda b, t: (b * nt0 + t, 0, 0),
                               memory_space=pltpu.SMEM),
                  _row_spec3(_TQ, cout, lambda b, t: (b * nt0 + t, 0, 0))],
        out_specs=_row_spec3(_TQ, cout, lambda b, t: (b * nt0 + t, 0, 0)),
        out_shape=jax.ShapeDtypeStruct((bs * p0, 1, cout), jnp.float32),
        compiler_params=cparams,
    )(g1.reshape(bs * p0, 1, cout),
      idx1.reshape(bs * nt0, _TQ, kn),
      h1.reshape(bs * p0, 1, cout))

    ntu = p1 // _TQ
    f_up, r_up = pl.pallas_call(
        _upsample_kernel,
        grid=(bs, ntu),
        in_specs=[_row_spec3(p0, cout, lambda b, t: (b, 0, 0)),
                  _row_spec3(p0, cout, lambda b, t: (b, 0, 0)),
                  pl.BlockSpec((1, _TQ, 3), lambda b, t: (b * ntu + t, 0, 0),
                               memory_space=pltpu.SMEM),
                  pl.BlockSpec((1, _TQ, 3), lambda b, t: (b * ntu + t, 0, 0),
                               memory_space=pltpu.SMEM)],
        out_specs=[_row_spec3(_TQ, cout, lambda b, t: (b * ntu + t, 0, 0))] * 2,
        out_shape=[jax.ShapeDtypeStruct((bs * p1, 1, cout), jnp.float32)] * 2,
        compiler_params=cparams,
    )(feats1,
      res1.reshape(bs * p0, 1, cout),
      idx3.reshape(bs * ntu, _TQ, 3),
      w3.reshape(bs * ntu, _TQ, 3))

    w2a, w2b = w2[:, :cout], w2[:, cout:]
    w2cat = jnp.concatenate([w2a - w2b, w2b], 0)
    h2, g2 = pl.pallas_call(
        _proj2_kernel,
        grid=(bs,),
        in_specs=[pl.BlockSpec((1, p1, cout), lambda b: (b, 0, 0)),
                  pl.BlockSpec((cout, 2 * cout), lambda b: (0, 0)),
                  pl.BlockSpec((1, cout), lambda b: (0, 0))],
        out_specs=[pl.BlockSpec((1, p1, cout), lambda b: (b, 0, 0))] * 2,
        out_shape=[jax.ShapeDtypeStruct((bs, p1, cout), jnp.float32)] * 2,
        compiler_params=cparams1,
    )(f_up.reshape(bs, p1, cout), w2cat.T, b2[None, :])

    out_rows = pl.pallas_call(
        _gmax2_kernel,
        grid=(bs, nt1),
        in_specs=[_row_spec3(p1, cout, lambda b, t: (b, 0, 0)),
                  pl.BlockSpec((1, _TQ, kn), lambda b, t: (b * nt1 + t, 0, 0),
                               memory_space=pltpu.SMEM),
                  _row_spec3(_TQ, cout, lambda b, t: (b * nt1 + t, 0, 0)),
                  _row_spec3(_TQ, cout, lambda b, t: (b * nt1 + t, 0, 0))],
        out_specs=_row_spec3(_TQ, cout, lambda b, t: (b * nt1 + t, 0, 0)),
        out_shape=jax.ShapeDtypeStruct((bs * p1, 1, cout), jnp.float32),
        compiler_params=cparams,
    )(g2.reshape(bs * p1, 1, cout),
      idx2.reshape(bs * nt1, _TQ, kn),
      h2.reshape(bs * p1, 1, cout),
      r_up)

    out = out_rows.reshape(bs, p1, cout).transpose(0, 2, 1)
    return parent_xyz, out


# speedup vs baseline: 1.1290x; 1.0021x over previous
import jax
import jax.numpy as jnp
from jax.experimental import pallas as pl
from jax.experimental.pallas import tpu as pltpu


def _proj1_kernel(f_ref, w_ref, b_ref, h_ref, res_ref, g_ref):
    cout = h_ref.shape[2]
    p = jnp.dot(f_ref[0], w_ref[...], preferred_element_type=jnp.float32)
    h_ref[0] = p[:, :cout] + b_ref[...]
    res_ref[0] = p[:, cout:2 * cout]
    g_ref[0] = p[:, 2 * cout:]


def _proj2_kernel(f_ref, w_ref, b_ref, h_ref, g_ref):
    cout = h_ref.shape[2]
    p = jnp.dot(f_ref[0], w_ref[...], preferred_element_type=jnp.float32)
    h_ref[0] = p[:, :cout] + b_ref[...]
    g_ref[0] = p[:, cout:]


def _gmax1_kernel(g_ref, idx_ref, h_ref, out_ref):
    kn = idx_ref.shape[2]

    def body(qb, carry):
        q0 = qb * 8
        accs = []
        for i in range(8):
            acc = g_ref[idx_ref[0, q0 + i, 0]]
            for k in range(1, kn):
                acc = jnp.maximum(acc, g_ref[idx_ref[0, q0 + i, k]])
            accs.append(acc)
        for i in range(8):
            out_ref[q0 + i] = jnp.maximum(h_ref[q0 + i] + accs[i], 0.0)
        return carry

    jax.lax.fori_loop(0, idx_ref.shape[1] // 8, body, 0)


def _gmax2_kernel(g_ref, idx_ref, h_ref, res_ref, out_ref):
    kn = idx_ref.shape[2]

    def body(qb, carry):
        q0 = qb * 8
        accs = []
        for i in range(8):
            acc = g_ref[idx_ref[0, q0 + i, 0]]
            for k in range(1, kn):
                acc = jnp.maximum(acc, g_ref[idx_ref[0, q0 + i, k]])
            accs.append(acc)
        for i in range(8):
            out_ref[q0 + i] = jnp.maximum(
                h_ref[q0 + i] + accs[i] + res_ref[q0 + i], 0.0)
        return carry

    jax.lax.fori_loop(0, idx_ref.shape[1] // 8, body, 0)


def _upsample_proj2_kernel(f_ref, r_ref, idx_ref, w_ref, w2_ref, b2_ref,
                           h2_ref, g2_ref, ro_ref, fo2d):
    cout = h2_ref.shape[2]

    def body(qb, carry):
        q0 = qb * 8
        fos, ros = [], []
        for i in range(8):
            i0 = idx_ref[0, q0 + i, 0]
            i1 = idx_ref[0, q0 + i, 1]
            i2 = idx_ref[0, q0 + i, 2]
            w0 = w_ref[0, q0 + i, 0]
            w1 = w_ref[0, q0 + i, 1]
            w2 = w_ref[0, q0 + i, 2]
            fos.append(f_ref[i0] * w0 + f_ref[i1] * w1 + f_ref[i2] * w2)
            ros.append(r_ref[i0] * w0 + r_ref[i1] * w1 + r_ref[i2] * w2)
        fo2d[pl.ds(q0, 8), :] = jnp.concatenate(fos, axis=0)
        for i in range(8):
            ro_ref[q0 + i] = ros[i]
        return carry

    jax.lax.fori_loop(0, idx_ref.shape[1] // 8, body, 0)

    p = jnp.dot(fo2d[...], w2_ref[...], preferred_element_type=jnp.float32)
    h2_ref[0] = p[:, :cout] + b2_ref[...]
    g2_ref[0] = p[:, cout:]


_TB = 256
_CH = 128


def _ballq_kernel(ptsT_ref, q_ref, out_ref, idx_acc, carry_ref, smin_ref):
    kn = out_ref.shape[2]
    tb = q_ref.shape[1]
    p = ptsT_ref.shape[2]

    idx_acc[...] = jnp.full((tb, kn), -1.0, jnp.float32)
    carry_ref[...] = jnp.zeros((tb, _CH), jnp.float32)
    smin_ref[0] = 0.0

    qx = q_ref[0, :, 0:1]
    qy = q_ref[0, :, 1:2]
    qz = q_ref[0, :, 2:3]

    lt = (jax.lax.broadcasted_iota(jnp.int32, (_CH, _CH), 0)
          <= jax.lax.broadcasted_iota(jnp.int32, (_CH, _CH), 1)
          ).astype(jnp.float32)
    lane_iota = jax.lax.broadcasted_iota(
        jnp.int32, (tb, _CH), 1).astype(jnp.float32)

    def body(chunk, carry_in):
        @pl.when(smin_ref[0] < kn)
        def _():
            base = pl.multiple_of(chunk * _CH, _CH)
            x0 = ptsT_ref[0, 0:1, pl.ds(base, _CH)]
            y0 = ptsT_ref[0, 1:2, pl.ds(base, _CH)]
            z0 = ptsT_ref[0, 2:3, pl.ds(base, _CH)]
            dx = qx - x0
            dy = qy - y0
            dz = qz - z0
            d2 = (dx * dx + dy * dy) + dz * dz
            within = (d2 < 0.5625).astype(jnp.float32)
            r = jnp.dot(within, lt, preferred_element_type=jnp.float32)
            carry = carry_ref[...]
            grank = jnp.where(within > 0.5, carry + r, 0.0)
            jglob = lane_iota + chunk.astype(jnp.float32) * _CH
            vals = [jnp.max(jnp.where(grank == (s + 1.0), jglob, -1.0),
                            axis=-1, keepdims=True) for s in range(kn)]
            contrib = jnp.concatenate(vals, axis=1)
            idx_acc[...] = jnp.maximum(idx_acc[...], contrib)
            new_carry = carry + jnp.sum(within, axis=-1, keepdims=True)
            carry_ref[...] = new_carry
            smin_ref[0] = jnp.min(new_carry)
        return carry_in

    jax.lax.fori_loop(0, p // _CH, body, 0)

    cnt = carry_ref[...][:, :kn]
    acc = idx_acc[...]
    slot = jax.lax.broadcasted_iota(jnp.int32, (tb, kn), 1).astype(jnp.float32)
    final = jnp.where(slot < cnt, acc, acc[:, 0:1])
    out_ref[0] = final.astype(jnp.int32)


def _three_nn_kernel(ptsT_ref, q_ref, idx_ref, w_ref, d2s_ref):
    tb = q_ref.shape[1]
    p = ptsT_ref.shape[2]
    nch = p // _CH

    qx = q_ref[0, :, 0:1]
    qy = q_ref[0, :, 1:2]
    qz = q_ref[0, :, 2:3]
    for chunk in range(nch):
        x0 = ptsT_ref[0, 0:1, chunk * _CH:(chunk + 1) * _CH]
        y0 = ptsT_ref[0, 1:2, chunk * _CH:(chunk + 1) * _CH]
        z0 = ptsT_ref[0, 2:3, chunk * _CH:(chunk + 1) * _CH]
        dx = qx - x0
        dy = qy - y0
        dz = qz - z0
        d2s_ref[:, chunk * _CH:(chunk + 1) * _CH] = (dx * dx + dy * dy) + dz * dz

    big = jnp.float32(3.0e38)
    found_i = []
    found_v = []
    for r in range(3):
        acc = None
        for chunk in range(nch):
            d2c = d2s_ref[:, chunk * _CH:(chunk + 1) * _CH]
            jglob = (jax.lax.broadcasted_iota(jnp.int32, (tb, _CH), 1)
                     .astype(jnp.float32) + jnp.float32(chunk * _CH))
            for fi in found_i:
                d2c = jnp.where(jglob == fi, big, d2c)
            acc = d2c if acc is None else jnp.minimum(acc, d2c)
        gmin = jnp.min(acc, axis=-1, keepdims=True)
        iacc = None
        for chunk in range(nch):
            d2c = d2s_ref[:, chunk * _CH:(chunk + 1) * _CH]
            jglob = (jax.lax.broadcasted_iota(jnp.int32, (tb, _CH), 1)
                     .astype(jnp.float32) + jnp.float32(chunk * _CH))
            for fi in found_i:
                d2c = jnp.where(jglob == fi, big, d2c)
            cand = jnp.where(d2c == gmin, jglob, big)
            iacc = cand if iacc is None else jnp.minimum(iacc, cand)
        gidx = jnp.min(iacc, axis=-1, keepdims=True)
        found_i.append(gidx)
        found_v.append(gmin)

    w = [1.0 / (v + 1e-8) for v in found_v]
    wsum = (w[0] + w[1]) + w[2]
    w = [x / wsum for x in w]
    idx_ref[0] = jnp.concatenate(found_i, axis=1).astype(jnp.int32)
    w_ref[0] = jnp.concatenate(w, axis=1)


def _ball_query_idx(ptsT, pts4, k):
    bs, _, p = ptsT.shape
    nt = p // _TB
    return pl.pallas_call(
        _ballq_kernel,
        grid=(bs, nt),
        in_specs=[pl.BlockSpec((1, 8, p), lambda b, t: (b, 0, 0)),
                  pl.BlockSpec((1, _TB, 4), lambda b, t: (b, t, 0))],
        out_specs=pl.BlockSpec((1, _TB, k), lambda b, t: (b, t, 0)),
        out_shape=jax.ShapeDtypeStruct((bs, p, k), jnp.int32),
        scratch_shapes=[pltpu.VMEM((_TB, k), jnp.float32),
                        pltpu.VMEM((_TB, _CH), jnp.float32),
                        pltpu.SMEM((1,), jnp.float32)],
        compiler_params=pltpu.CompilerParams(
            dimension_semantics=("parallel", "parallel")),
    )(ptsT, pts4)


def _three_nn(ptsT_child, pts4_parent):
    bs, _, pc = ptsT_child.shape
    pp = pts4_parent.shape[1]
    nt = pp // _TB
    return pl.pallas_call(
        _three_nn_kernel,
        grid=(bs, nt),
        in_specs=[pl.BlockSpec((1, 8, pc), lambda b, t: (b, 0, 0)),
                  pl.BlockSpec((1, _TB, 4), lambda b, t: (b, t, 0))],
        out_specs=[pl.BlockSpec((1, _TB, 3), lambda b, t: (b, t, 0)),
                   pl.BlockSpec((1, _TB, 3), lambda b, t: (b, t, 0))],
        out_shape=[jax.ShapeDtypeStruct((bs, pp, 3), jnp.int32),
                   jax.ShapeDtypeStruct((bs, pp, 3), jnp.float32)],
        scratch_shapes=[pltpu.VMEM((_TB, pc), jnp.float32)],
        compiler_params=pltpu.CompilerParams(
            dimension_semantics=("parallel", "parallel")),
    )(ptsT_child, pts4_parent)


def _coord_views(pts):
    ptsT = jnp.pad(pts.transpose(0, 2, 1), ((0, 0), (0, 5), (0, 0)))
    pts4 = jnp.pad(pts, ((0, 0), (0, 0), (0, 1)))
    return ptsT, pts4


_TQ = 256


def _row_spec3(nrows, cols, index_map):
    return pl.BlockSpec((nrows, 1, cols), index_map)


def kernel(xyz, parent_xyz, feats, w1, b1, w2, b2, w_res):
    bs, c, p0 = feats.shape
    p1 = parent_xyz.shape[1]
    cout = w1.shape[0]
    kn = 32
    nt0 = p0 // _TQ
    nt1 = p1 // _TQ

    cparams = pltpu.CompilerParams(
        dimension_semantics=("parallel", "parallel"))
    cparams1 = pltpu.CompilerParams(dimension_semantics=("parallel",))

    xyzT, xyz4 = _coord_views(xyz)
    pxyzT, pxyz4 = _coord_views(parent_xyz)
    idx1 = _ball_query_idx(xyzT, xyz4, kn)
    idx2 = _ball_query_idx(pxyzT, pxyz4, kn)
    idx3, w3 = _three_nn(xyzT, pxyz4)

    feats_t = feats.transpose(0, 2, 1)
    w1a, w1b = w1[:, :c], w1[:, c:]
    w1cat = jnp.concatenate([w1a - w1b, w_res, w1b], 0)
    h1, res1, g1 = pl.pallas_call(
        _proj1_kernel,
        grid=(bs,),
        in_specs=[pl.BlockSpec((1, p0, c), lambda b: (b, 0, 0)),
                  pl.BlockSpec((c, 3 * cout), lambda b: (0, 0)),
                  pl.BlockSpec((1, cout), lambda b: (0, 0))],
        out_specs=[pl.BlockSpec((1, p0, cout), lambda b: (b, 0, 0))] * 3,
        out_shape=[jax.ShapeDtypeStruct((bs, p0, cout), jnp.float32)] * 3,
        compiler_params=cparams1,
    )(feats_t, w1cat.T, b1[None, :])

    feats1 = pl.pallas_call(
        _gmax1_kernel,
        grid=(bs, nt0),
        in_specs=[_row_spec3(p0, cout, lambda b, t: (b, 0, 0)),
                  pl.BlockSpec((1, _TQ, kn), lambda b, t: (b * nt0 + t, 0, 0),
                               memory_space=pltpu.SMEM),
                  _row_spec3(_TQ, cout, lambda b, t: (b * nt0 + t, 0, 0))],
        out_specs=_row_spec3(_TQ, cout, lambda b, t: (b * nt0 + t, 0, 0)),
        out_shape=jax.ShapeDtypeStruct((bs * p0, 1, cout), jnp.float32),
        compiler_params=cparams,
    )(g1.reshape(bs * p0, 1, cout),
      idx1.reshape(bs * nt0, _TQ, kn),
      h1.reshape(bs * p0, 1, cout))

    ntu = p1 // _TQ
    w2a, w2b = w2[:, :cout], w2[:, cout:]
    w2cat = jnp.concatenate([w2a - w2b, w2b], 0)
    h2, g2, r_up = pl.pallas_call(
        _upsample_proj2_kernel,
        grid=(bs, ntu),
        in_specs=[_row_spec3(p0, cout, lambda b, t: (b, 0, 0)),
                  _row_spec3(p0, cout, lambda b, t: (b, 0, 0)),
                  pl.BlockSpec((1, _TQ, 3), lambda b, t: (b * ntu + t, 0, 0),
                               memory_space=pltpu.SMEM),
                  pl.BlockSpec((1, _TQ, 3), lambda b, t: (b * ntu + t, 0, 0),
                               memory_space=pltpu.SMEM),
                  pl.BlockSpec((cout, 2 * cout), lambda b, t: (0, 0)),
                  pl.BlockSpec((1, cout), lambda b, t: (0, 0))],
        out_specs=[pl.BlockSpec((1, _TQ, cout),
                                lambda b, t: (b * ntu + t, 0, 0)),
                   pl.BlockSpec((1, _TQ, cout),
                                lambda b, t: (b * ntu + t, 0, 0)),
                   _row_spec3(_TQ, cout, lambda b, t: (b * ntu + t, 0, 0))],
        out_shape=[jax.ShapeDtypeStruct((bs * ntu, _TQ, cout), jnp.float32),
                   jax.ShapeDtypeStruct((bs * ntu, _TQ, cout), jnp.float32),
                   jax.ShapeDtypeStruct((bs * p1, 1, cout), jnp.float32)],
        scratch_shapes=[pltpu.VMEM((_TQ, cout), jnp.float32)],
        compiler_params=cparams,
    )(feats1,
      res1.reshape(bs * p0, 1, cout),
      idx3.reshape(bs * ntu, _TQ, 3),
      w3.reshape(bs * ntu, _TQ, 3),
      w2cat.T, b2[None, :])

    out_rows = pl.pallas_call(
        _gmax2_kernel,
        grid=(bs, nt1),
        in_specs=[_row_spec3(p1, cout, lambda b, t: (b, 0, 0)),
                  pl.BlockSpec((1, _TQ, kn), lambda b, t: (b * nt1 + t, 0, 0),
                               memory_space=pltpu.SMEM),
                  _row_spec3(_TQ, cout, lambda b, t: (b * nt1 + t, 0, 0)),
                  _row_spec3(_TQ, cout, lambda b, t: (b * nt1 + t, 0, 0))],
        out_specs=_row_spec3(_TQ, cout, lambda b, t: (b * nt1 + t, 0, 0)),
        out_shape=jax.ShapeDtypeStruct((bs * p1, 1, cout), jnp.float32),
        compiler_params=cparams,
    )(g2.reshape(bs * p1, 1, cout),
      idx2.reshape(bs * nt1, _TQ, kn),
      h2.reshape(bs * p1, 1, cout),
      r_up)

    out = out_rows.reshape(bs, p1, cout).transpose(0, 2, 1)
    return parent_xyz, out
